# baseline (speedup 1.0000x reference)
import jax
import jax.numpy as jnp
from jax import lax
from jax.experimental import pallas as pl

TOP_K = 8
N_EXPERTS = 64
BLOCK_ROWS = 1024


def _gate_kernel(x_ref, w_ref, idx_ref, wt_ref):
    x = x_ref[...]
    w = w_ref[...]
    s = lax.dot_general(w, x, (((1,), (1,)), ((), ())),
                        preferred_element_type=jnp.float32)
    iota0 = lax.broadcasted_iota(jnp.int32, s.shape, 0)
    neg_inf = jnp.float32(-jnp.inf)
    vals, idxs = [], []
    for _ in range(TOP_K):
        m = jnp.max(s, axis=0, keepdims=True)
        i = jnp.min(jnp.where(s == m, iota0, N_EXPERTS), axis=0, keepdims=True)
        vals.append(m)
        idxs.append(i)
        s = jnp.where(iota0 == i, neg_inf, s)
    v = jnp.concatenate(vals, axis=0)
    ii = jnp.concatenate(idxs, axis=0)
    e = jnp.exp(v - v[0:1])
    wt = e / jnp.sum(e, axis=0, keepdims=True)
    eyek = jnp.eye(TOP_K, dtype=jnp.float32)
    wt_t = lax.dot_general(wt, eyek, (((0,), (0,)), ((), ())),
                           preferred_element_type=jnp.float32)
    idx_t = lax.dot_general(ii.astype(jnp.float32), eyek,
                            (((0,), (0,)), ((), ())),
                            preferred_element_type=jnp.float32)
    idx_ref[...] = idx_t.astype(jnp.int32)
    wt_ref[...] = wt_t


def kernel(hidden_states, weight):
    bsz, seq_len, h = hidden_states.shape
    x = hidden_states.reshape(-1, h)
    n = x.shape[0]
    grid = (n // BLOCK_ROWS,)
    idx, wt = pl.pallas_call(
        _gate_kernel,
        grid=grid,
        in_specs=[
            pl.BlockSpec((BLOCK_ROWS, h), lambda r: (r, 0)),
            pl.BlockSpec((N_EXPERTS, h), lambda r: (0, 0)),
        ],
        out_specs=[
            pl.BlockSpec((BLOCK_ROWS, TOP_K), lambda r: (r, 0)),
            pl.BlockSpec((BLOCK_ROWS, TOP_K), lambda r: (r, 0)),
        ],
        out_shape=[
            jax.ShapeDtypeStruct((n, TOP_K), jnp.int32),
            jax.ShapeDtypeStruct((n, TOP_K), jnp.float32),
        ],
    )(x, weight)
    return idx, wt


# speedup vs baseline: 1.0282x; 1.0282x over previous
import jax
import jax.numpy as jnp
from jax import lax
from jax.experimental import pallas as pl

TOP_K = 8
N_EXPERTS = 64
BLOCK_ROWS = 2048


def _gate_kernel(x_ref, w_ref, idx_ref, wt_ref):
    x = x_ref[...]
    w = w_ref[...]
    s = lax.dot_general(w, x, (((1,), (1,)), ((), ())),
                        preferred_element_type=jnp.float32)
    iota0 = lax.broadcasted_iota(jnp.int32, s.shape, 0)
    neg_inf = jnp.float32(-jnp.inf)
    vals, idxs = [], []
    for _ in range(TOP_K):
        m = jnp.max(s, axis=0, keepdims=True)
        i = jnp.min(jnp.where(s == m, iota0, N_EXPERTS), axis=0, keepdims=True)
        vals.append(m)
        idxs.append(i)
        s = jnp.where(iota0 == i, neg_inf, s)
    v = jnp.concatenate(vals, axis=0)
    ii = jnp.concatenate(idxs, axis=0)
    e = jnp.exp(v - v[0:1])
    wt = e / jnp.sum(e, axis=0, keepdims=True)
    eyek = jnp.eye(TOP_K, dtype=jnp.float32)
    wt_t = lax.dot_general(wt, eyek, (((0,), (0,)), ((), ())),
                           preferred_element_type=jnp.float32)
    idx_t = lax.dot_general(ii.astype(jnp.float32), eyek,
                            (((0,), (0,)), ((), ())),
                            preferred_element_type=jnp.float32)
    idx_ref[...] = idx_t.astype(jnp.int32)
    wt_ref[...] = wt_t


def kernel(hidden_states, weight):
    bsz, seq_len, h = hidden_states.shape
    x = hidden_states.reshape(-1, h)
    n = x.shape[0]
    grid = (n // BLOCK_ROWS,)
    idx, wt = pl.pallas_call(
        _gate_kernel,
        grid=grid,
        in_specs=[
            pl.BlockSpec((BLOCK_ROWS, h), lambda r: (r, 0)),
            pl.BlockSpec((N_EXPERTS, h), lambda r: (0, 0)),
        ],
        out_specs=[
            pl.BlockSpec((BLOCK_ROWS, TOP_K), lambda r: (r, 0)),
            pl.BlockSpec((BLOCK_ROWS, TOP_K), lambda r: (r, 0)),
        ],
        out_shape=[
            jax.ShapeDtypeStruct((n, TOP_K), jnp.int32),
            jax.ShapeDtypeStruct((n, TOP_K), jnp.float32),
        ],
    )(x, weight)
    return idx, wt
